# baseline (speedup 1.0000x reference)
import functools

import jax
import jax.numpy as jnp
from jax import lax
from jax.experimental import pallas as pl
from jax.experimental.pallas import tpu as pltpu
from jax.experimental.pallas import tpu_sc as plsc

_DEFAULT_VALUE = 0

_L = 16
_NC = 2
_NS = 16
_NW = _NC * _NS

_R = 16384
_C = 26
_ROWS_W = _R // _NW
_UNROLL = 8


def _lookup_sc(a, key16, val16):
    mesh = plsc.VectorSubcoreMesh(core_axis_name="c", subcore_axis_name="s")

    @functools.partial(
        pl.kernel,
        mesh=mesh,
        out_type=jax.ShapeDtypeStruct((_R, _C), jnp.int32),
        scratch_types=[
            pltpu.VMEM((_ROWS_W, _C), jnp.int32),
            pltpu.VMEM((_L,), jnp.int32),
            pltpu.VMEM((_L,), jnp.int32),
            pltpu.SemaphoreType.DMA,
        ],
    )
    def _k(a_hbm, key_hbm, val_hbm, out_hbm, a_v, key_v, val_v, sem):
        wid = lax.axis_index("s") * _NC + lax.axis_index("c")
        r_base = wid * _ROWS_W
        c_a = pltpu.async_copy(a_hbm.at[pl.ds(r_base, _ROWS_W)], a_v, sem)
        c_k = pltpu.async_copy(key_hbm, key_v, sem)
        c_v = pltpu.async_copy(val_hbm, val_v, sem)
        c_a.wait()
        c_k.wait()
        c_v.wait()
        key = key_v[...]
        val = val_v[...]
        default = jnp.full((_L,), _DEFAULT_VALUE, jnp.int32)

        def do(r):
            x0 = a_v[r, pl.ds(0, _L)]
            x1 = a_v[r, pl.ds(_C - _L, _L)]
            a_v[r, pl.ds(0, _L)] = jnp.where(x0 == key, val, default)
            a_v[r, pl.ds(_C - _L, _L)] = jnp.where(x1 == key, val, default)

        def body(i, carry):
            r0 = i * _UNROLL
            for u in range(_UNROLL):
                do(r0 + u)
            return carry

        lax.fori_loop(0, _ROWS_W // _UNROLL, body, 0)
        pltpu.sync_copy(a_v, out_hbm.at[pl.ds(r_base, _ROWS_W)])

    return _k(a, key16, val16)


def kernel(a, table_keys, table_values):
    a2 = a.astype(jnp.int32)
    key16 = jnp.broadcast_to(table_keys.astype(jnp.int32), (_L,))
    val16 = jnp.broadcast_to(table_values.astype(jnp.int32), (_L,))
    return {"y_click": _lookup_sc(a2, key16, val16)}


# speedup vs baseline: 1.6604x; 1.6604x over previous
import jax
import jax.numpy as jnp
from jax.experimental import pallas as pl
from jax.experimental.pallas import tpu as pltpu

_DEFAULT_VALUE = 0

_R = 16384
_C = 26
_BLK = 2048
_GRID = _R // _BLK


def _body(key_ref, val_ref, a_ref, o_ref):
    key = key_ref[0]
    val = val_ref[0]
    x = a_ref[...]
    o_ref[...] = jnp.where(x == key, val, jnp.int32(_DEFAULT_VALUE))


def kernel(a, table_keys, table_values):
    a2 = a.astype(jnp.int32)
    key = table_keys.astype(jnp.int32)
    val = table_values.astype(jnp.int32)
    out = pl.pallas_call(
        _body,
        grid=(_GRID,),
        in_specs=[
            pl.BlockSpec(memory_space=pltpu.SMEM),
            pl.BlockSpec(memory_space=pltpu.SMEM),
            pl.BlockSpec((_BLK, _C), lambda i: (i, 0)),
        ],
        out_specs=pl.BlockSpec((_BLK, _C), lambda i: (i, 0)),
        out_shape=jax.ShapeDtypeStruct((_R, _C), jnp.int32),
    )(key, val, a2)
    return {"y_click": out}
